# baseline (speedup 1.0000x reference)
import functools

import jax
import jax.numpy as jnp
from jax import lax
from jax.experimental import pallas as pl
from jax.experimental.pallas import tpu as pltpu
from jax.experimental.pallas import tpu_sc as plsc

BATCH = 4096
SEQ = 200
DIM = 64
NUM_CLASSES = 5
NC = 2
NS = 16
NW = NC * NS
B_PER_W = BATCH // NW
SPLIT = 104
CPAD = 16


def _make_sc_call():
  mesh = plsc.VectorSubcoreMesh(core_axis_name="c", subcore_axis_name="s")

  @functools.partial(
      pl.kernel,
      mesh=mesh,
      compiler_params=pltpu.CompilerParams(use_tc_tiling_on_sc=False),
      out_type=jax.ShapeDtypeStruct((BATCH, CPAD), jnp.float32),
      scratch_types=[
          pltpu.VMEM((B_PER_W * SEQ,), jnp.int32),
          pltpu.VMEM((NUM_CLASSES, DIM), jnp.float32),
          pltpu.VMEM((CPAD,), jnp.float32),
          pltpu.VMEM((SEQ, DIM), jnp.float32),
          pltpu.VMEM((SEQ, DIM), jnp.float32),
          pltpu.VMEM((B_PER_W, CPAD), jnp.float32),
          pltpu.SemaphoreType.DMA,
          pltpu.SemaphoreType.DMA,
      ],
  )
  def sc_call(x_hbm, table_hbm, w_hbm, b_hbm, out_hbm,
              x_v, w_v, b_v, buf0, buf1, out_v, sem0, sem1):
    wid = lax.axis_index("s") * NC + lax.axis_index("c")
    base = wid * B_PER_W

    pltpu.sync_copy(x_hbm.at[pl.ds(base * SEQ, B_PER_W * SEQ)], x_v)
    pltpu.sync_copy(w_hbm, w_v)
    pltpu.sync_copy(b_hbm, b_v)

    def issue(row, buf, sem):
      pltpu.async_copy(
          table_hbm.at[x_v.at[pl.ds(row * SEQ, SPLIT)]],
          buf.at[pl.ds(0, SPLIT)], sem)
      pltpu.async_copy(
          table_hbm.at[x_v.at[pl.ds(row * SEQ + SPLIT, SEQ - SPLIT)]],
          buf.at[pl.ds(SPLIT, SEQ - SPLIT)], sem)

    def wait(row, buf, sem):
      pltpu.make_async_copy(
          table_hbm.at[x_v.at[pl.ds(row * SEQ, SPLIT)]],
          buf.at[pl.ds(0, SPLIT)], sem).wait()
      pltpu.make_async_copy(
          table_hbm.at[x_v.at[pl.ds(row * SEQ + SPLIT, SEQ - SPLIT)]],
          buf.at[pl.ds(SPLIT, SEQ - SPLIT)], sem).wait()

    lanes = lax.iota(jnp.int32, 16)

    def compute(row, buf):
      acc = tuple(buf[0, pl.ds(16 * j, 16)] for j in range(4))

      def mx(i, a):
        return tuple(
            jnp.maximum(a[j], buf[i, pl.ds(16 * j, 16)]) for j in range(4))

      acc = lax.fori_loop(1, SEQ, mx, acc, unroll=4)

      yv = b_v[...]
      for c in range(NUM_CLASSES):
        t = acc[0] * w_v[c, pl.ds(0, 16)]
        for j in range(1, 4):
          t = t + acc[j] * w_v[c, pl.ds(16 * j, 16)]
        for sh in (8, 4, 2, 1):
          t = t + t.at[lanes ^ sh].get(mode="promise_in_bounds",
                                       unique_indices=True)
        yv = jnp.where(lanes == c, yv + t, yv)
      out_v[row, pl.ds(0, CPAD)] = yv

    issue(0, buf0, sem0)
    bufs = ((buf0, sem0), (buf1, sem1))

    def step(r, _):
      for par, (buf, sem) in enumerate(bufs):
        row = r + par
        nbuf, nsem = bufs[(par + 1) % 2]

        @pl.when(row + 1 < B_PER_W)
        def _():
          issue(row + 1, nbuf, nsem)

        wait(row, buf, sem)
        compute(row, buf)
      return 0

    lax.fori_loop(0, B_PER_W // 2, lambda i, c: step(i * 2, c), 0)

    pltpu.sync_copy(out_v, out_hbm.at[pl.ds(base, B_PER_W)])

  return sc_call


_sc_call = _make_sc_call()


@jax.jit
def kernel(x, table, W, b):
  b_pad = jnp.zeros((CPAD,), jnp.float32).at[:NUM_CLASSES].set(b)
  out = _sc_call(x.astype(jnp.int32).reshape(-1), table, W, b_pad)
  return out[:, :NUM_CLASSES]
